# baseline (speedup 1.0000x reference)
import functools

import jax
import jax.numpy as jnp
from jax import lax
from jax.experimental import pallas as pl
from jax.experimental.pallas import tpu as pltpu
from jax.experimental.pallas import tpu_sc as plsc

N = 10000
E = 160000
DIM = 256
H = 8
HD = 32
HALF = 128
NC = 2
NS = 16
CH = 32
BLKE = 640
CPB = BLKE // CH
NBLK = E // BLKE
NPAD = 10112
ROWS_PER_TILE = NPAD // NS
DW = 1280
DEN_N = 10240
SCALE = 1.0 / (HD ** 0.5)


def _qkv_tc(x, Wcat, bcat):
    BM = 1000

    def body(x_ref, w_ref, b_ref, q_ref, k_ref, v_ref):
        y = jnp.dot(x_ref[...], w_ref[...],
                    preferred_element_type=jnp.float32) + b_ref[...]
        q_ref[0] = y[:, 0:128]
        q_ref[1] = y[:, 128:256]
        k_ref[0] = y[:, 256:384]
        k_ref[1] = y[:, 384:512]
        v_ref[0] = y[:, 512:640]
        v_ref[1] = y[:, 640:768]

    return pl.pallas_call(
        body,
        grid=(N // BM,),
        in_specs=[
            pl.BlockSpec((BM, DIM), lambda i: (i, 0)),
            pl.BlockSpec((DIM, 3 * DIM), lambda i: (0, 0)),
            pl.BlockSpec((1, 3 * DIM), lambda i: (0, 0)),
        ],
        out_specs=[pl.BlockSpec((NC, BM, HALF), lambda i: (0, i, 0))] * 3,
        out_shape=[jax.ShapeDtypeStruct((NC, N, HALF), jnp.float32)] * 3,
    )(x, Wcat, bcat)


def _sc_attn(qh, kh, vh, edge_index):
    mesh = plsc.VectorSubcoreMesh(core_axis_name="c", subcore_axis_name="s")

    @functools.partial(
        pl.kernel,
        out_type=(
            jax.ShapeDtypeStruct((NC * NPAD, HALF), jnp.float32),
            jax.ShapeDtypeStruct((NC * DEN_N, 16), jnp.float32),
        ),
        mesh=mesh,
        scratch_types=[
            pltpu.VMEM((2, BLKE), jnp.int32),
            pltpu.VMEM((4, CH), jnp.int32),
            pltpu.VMEM((4, CH), jnp.int32),
            pltpu.VMEM((CH, HALF), jnp.float32),
            pltpu.VMEM((CH, HALF), jnp.float32),
            pltpu.VMEM((CH, HALF), jnp.float32),
            pltpu.VMEM((CH, HALF), jnp.float32),
            pltpu.VMEM((CH, HALF), jnp.float32),
            pltpu.VMEM((CH, 16), jnp.float32),
            pltpu.VMEM((CH, HALF), jnp.float32),
            pltpu.VMEM((8, 16), jnp.float32),
            pltpu.SemaphoreType.DMA,
            pltpu.SemaphoreType.DMA,
            pltpu.SemaphoreType.DMA,
            pltpu.SemaphoreType.DMA,
            pltpu.VMEM_SHARED((NPAD, HALF), jnp.float32),
            pltpu.VMEM_SHARED((DW, HALF), jnp.float32),
        ],
    )
    def attn(q_hbm, k_hbm, v_hbm, ei_hbm, acc_out, den_out,
             blk, idxA, idxB, kA, kB, qA, qB, v_rows, wbuf, dbuf, den_nar,
             gsemA, gsemB, vsem, ssem, acc_sh, den_wide):
        c = lax.axis_index("c")
        s = lax.axis_index("s")
        zero16 = jnp.zeros((16,), jnp.float32)

        def zbody(t, carry):
            for j in range(HALF // 16):
                v_rows[t, pl.ds(16 * j, 16)] = zero16
                dbuf[t, pl.ds(16 * j, 16)] = zero16
            wbuf[t, :] = zero16
            return carry

        lax.fori_loop(0, CH, zbody, 0)

        r0 = s * ROWS_PER_TILE
        for b in range(19):
            pltpu.sync_copy(v_rows, acc_sh.at[pl.ds(r0 + b * CH, CH), :])
        pltpu.sync_copy(v_rows, acc_sh.at[pl.ds(r0 + ROWS_PER_TILE - CH, CH), :])
        w0 = s * (DW // NS)
        pltpu.sync_copy(v_rows, den_wide.at[pl.ds(w0, CH), :])
        pltpu.sync_copy(v_rows, den_wide.at[pl.ds(w0 + CH, CH), :])
        pltpu.sync_copy(v_rows, den_wide.at[pl.ds(w0 + DW // NS - CH, CH), :])
        plsc.subcore_barrier()

        lane = lax.broadcasted_iota(jnp.int32, (16,), 0)
        head_mask = lane < 4
        shuf = [lane ^ m for m in (8, 4, 2, 1)]
        crow = c * N

        def hsum(p):
            for ix in shuf:
                p = p + jnp.take_along_axis(p, ix, axis=0)
            return p

        nblk = jnp.where(s < NBLK - 15 * NS, 16, 15)
        blk0 = s * 15 + jnp.minimum(s, NBLK - 15 * NS)
        nch = nblk * CPB

        def load_block(jj):
            ebase = pl.multiple_of(blk0 * BLKE + jj * CH, 128)
            pltpu.sync_copy(ei_hbm.at[:, pl.ds(ebase, BLKE)], blk)

        def prep_idx(jj, P):
            pos = (jj % CPB) * CH
            for t in range(CH // 16):
                sl = pl.ds(pos + 16 * t, 16)
                osl = pl.ds(16 * t, 16)
                sv = blk[0, sl]
                dv = blk[1, sl]
                P[0, osl] = dv
                P[1, osl] = sv + crow
                P[2, osl] = dv + crow
                P[3, osl] = lax.shift_right_logical(dv, 3)
            return None

        def scores(kb, qb):
            def edge_a(i, cc):
                total = jnp.zeros((16,), jnp.float32)
                for hj in range(4):
                    k0 = kb[i, pl.ds(32 * hj, 16)]
                    k1 = kb[i, pl.ds(32 * hj + 16, 16)]
                    q0 = qb[i, pl.ds(32 * hj, 16)]
                    q1 = qb[i, pl.ds(32 * hj + 16, 16)]
                    p = hsum(k0 * q0 + k1 * q1)
                    total = jnp.where(lane == hj, p, total)
                wbuf[i, :] = jnp.where(head_mask,
                                       jnp.exp(total * SCALE), 0.0)
                return cc

            lax.fori_loop(0, CH, edge_a, 0)

        def weight_scatter(P, qb):
            offs = []
            for g in range(CH // 16):
                offv = (P[0, pl.ds(16 * g, 16)] & 7) * 16
                for jj in range(16):
                    i = 16 * g + jj
                    wrow = wbuf[i, :]
                    for hj in range(4):
                        w = wrow[hj]
                        vr0 = v_rows[i, pl.ds(32 * hj, 16)]
                        vr1 = v_rows[i, pl.ds(32 * hj + 16, 16)]
                        v_rows[i, pl.ds(32 * hj, 16)] = vr0 * w
                        v_rows[i, pl.ds(32 * hj + 16, 16)] = vr1 * w
                    off = pl.multiple_of(offv[jj], 16)
                    offs.append(off)
                    dbuf[i, pl.ds(off, 16)] = wrow
            s1 = pltpu.async_copy(v_rows, acc_sh.at[P.at[0]], ssem, add=True)
            s2 = pltpu.async_copy(dbuf, den_wide.at[P.at[3]], ssem, add=True)
            s1.wait()
            s2.wait()
            for i in range(CH):
                dbuf[i, pl.ds(offs[i], 16)] = zero16

        load_block(0)
        prep_idx(0, idxA)
        pltpu.async_copy(k_hbm.at[idxA.at[1]], kA, gsemA)
        pltpu.async_copy(q_hbm.at[idxA.at[2]], qA, gsemA)

        def pair_body(j2, carry):
            jA = 2 * j2
            prep_idx(jA + 1, idxB)
            pltpu.async_copy(k_hbm.at[idxB.at[1]], kB, gsemB)
            pltpu.async_copy(q_hbm.at[idxB.at[2]], qB, gsemB)
            pltpu.async_copy(v_hbm.at[idxA.at[1]], v_rows, vsem)
            pltpu.make_async_copy(k_hbm.at[idxA.at[1]], kA, gsemA).wait()
            pltpu.make_async_copy(q_hbm.at[idxA.at[2]], qA, gsemA).wait()
            scores(kA, qA)
            pltpu.make_async_copy(v_hbm.at[idxA.at[1]], v_rows, vsem).wait()
            weight_scatter(idxA, qA)
            jN = jA + 2

            @pl.when(jN % CPB == 0)
            def _():
                @pl.when(jN < nch)
                def _():
                    load_block(jN)

            @pl.when(jN < nch)
            def _():
                prep_idx(jN, idxA)

            pltpu.async_copy(k_hbm.at[idxA.at[1]], kA, gsemA)
            pltpu.async_copy(q_hbm.at[idxA.at[2]], qA, gsemA)
            pltpu.async_copy(v_hbm.at[idxB.at[1]], v_rows, vsem)
            pltpu.make_async_copy(k_hbm.at[idxB.at[1]], kB, gsemB).wait()
            pltpu.make_async_copy(q_hbm.at[idxB.at[2]], qB, gsemB).wait()
            scores(kB, qB)
            pltpu.make_async_copy(v_hbm.at[idxB.at[1]], v_rows, vsem).wait()
            weight_scatter(idxB, qB)
            return carry

        lax.fori_loop(0, nch // 2, pair_body, 0)
        pltpu.make_async_copy(k_hbm.at[idxA.at[1]], kA, gsemA).wait()
        pltpu.make_async_copy(q_hbm.at[idxA.at[2]], qA, gsemA).wait()
        plsc.subcore_barrier()

        for b in range(19):
            pltpu.sync_copy(acc_sh.at[pl.ds(r0 + b * CH, CH), :], v_rows)
            pltpu.sync_copy(v_rows,
                            acc_out.at[pl.ds(c * NPAD + r0 + b * CH, CH), :])
        tb = ROWS_PER_TILE - CH
        pltpu.sync_copy(acc_sh.at[pl.ds(r0 + tb, CH), :], v_rows)
        pltpu.sync_copy(v_rows, acc_out.at[pl.ds(c * NPAD + r0 + tb, CH), :])

        for b in range(5):
            pltpu.sync_copy(den_wide.at[pl.ds(w0 + 16 * b, 16), :],
                            v_rows.at[pl.ds(0, 16), :])
            for r in range(16):
                for p8 in range(8):
                    den_nar[p8, :] = v_rows[r, pl.ds(16 * p8, 16)]
                pltpu.sync_copy(
                    den_nar,
                    den_out.at[pl.ds(c * DEN_N + 640 * s + 128 * b + 8 * r, 8), :])

    acc, den = attn(qh.reshape(NC * N, HALF), kh.reshape(NC * N, HALF),
                    vh.reshape(NC * N, HALF), edge_index)
    return acc.reshape(NC, NPAD, HALF), den.reshape(NC, DEN_N, 16)


def _out_tc(acc, den, W_out, b_out):
    BM = 1000

    def body(acc_ref, den_ref, w_ref, b_ref, o_ref):
        lane16 = lax.broadcasted_iota(jnp.int32, (BM, 16), 1)
        rows = lax.broadcasted_iota(jnp.int32, (16, HALF), 0)
        cols = lax.broadcasted_iota(jnp.int32, (16, HALF), 1)
        S = jnp.where(cols // HD == rows, 1.0, 0.0)
        out = None
        for cpart in range(NC):
            d = den_ref[cpart]
            r = jnp.where((d > 0) & (lane16 < 4), 1.0 / d, 0.0)
            rexp = jnp.dot(r, S, preferred_element_type=jnp.float32)
            a = acc_ref[cpart] * rexp
            wslice = w_ref[cpart * HALF:(cpart + 1) * HALF, :]
            t = jnp.dot(a, wslice, preferred_element_type=jnp.float32)
            out = t if out is None else out + t
        o_ref[...] = out + b_ref[...]

    return pl.pallas_call(
        body,
        grid=(N // BM,),
        in_specs=[
            pl.BlockSpec((NC, BM, HALF), lambda i: (0, i, 0)),
            pl.BlockSpec((NC, BM, 16), lambda i: (0, i, 0)),
            pl.BlockSpec((DIM, DIM), lambda i: (0, 0)),
            pl.BlockSpec((1, DIM), lambda i: (0, 0)),
        ],
        out_specs=pl.BlockSpec((BM, DIM), lambda i: (i, 0)),
        out_shape=jax.ShapeDtypeStruct((N, DIM), jnp.float32),
    )(acc, den, W_out, b_out)


def kernel(x, edge_index, W_qkv, b_qkv, W_out, b_out):
    Wr = W_qkv.reshape(DIM, H, 3 * HD)
    Wcat = jnp.concatenate([
        Wr[:, :, :HD].reshape(DIM, DIM),
        Wr[:, :, HD:2 * HD].reshape(DIM, DIM),
        Wr[:, :, 2 * HD:].reshape(DIM, DIM),
    ], axis=1)
    br = b_qkv.reshape(H, 3 * HD)
    bcat = jnp.concatenate([
        br[:, :HD].reshape(DIM),
        br[:, HD:2 * HD].reshape(DIM),
        br[:, 2 * HD:].reshape(DIM),
    ]).reshape(1, 3 * DIM)
    q, k, v = _qkv_tc(x, Wcat, bcat)
    acc, den = _sc_attn(q, k, v, edge_index)
    return _out_tc(acc, den, W_out, b_out.reshape(1, DIM))


# speedup vs baseline: 1.3181x; 1.3181x over previous
import functools

import jax
import jax.numpy as jnp
from jax import lax
from jax.experimental import pallas as pl
from jax.experimental.pallas import tpu as pltpu
from jax.experimental.pallas import tpu_sc as plsc

N = 10000
E = 160000
DIM = 256
H = 8
HD = 32
HALF = 128
NC = 2
NS = 16
CH = 32
BLKE = 640
CPB = BLKE // CH
NBLK = E // BLKE
NPAD = 10112
ROWS_PER_TILE = NPAD // NS
DW = 1280
DEN_N = 10240
SCALE = 1.0 / (HD ** 0.5)


def _qkv_tc(x, Wcat, bcat):
    BM = 1000

    def body(x_ref, w_ref, b_ref, q_ref, k_ref, v_ref):
        y = jnp.dot(x_ref[...], w_ref[...],
                    preferred_element_type=jnp.float32) + b_ref[...]
        q_ref[0] = y[:, 0:128]
        q_ref[1] = y[:, 128:256]
        k_ref[0] = y[:, 256:384]
        k_ref[1] = y[:, 384:512]
        v_ref[0] = y[:, 512:640]
        v_ref[1] = y[:, 640:768]

    return pl.pallas_call(
        body,
        grid=(N // BM,),
        in_specs=[
            pl.BlockSpec((BM, DIM), lambda i: (i, 0)),
            pl.BlockSpec((DIM, 3 * DIM), lambda i: (0, 0)),
            pl.BlockSpec((1, 3 * DIM), lambda i: (0, 0)),
        ],
        out_specs=[pl.BlockSpec((NC, BM, HALF), lambda i: (0, i, 0))] * 3,
        out_shape=[jax.ShapeDtypeStruct((NC, N, HALF), jnp.float32)] * 3,
    )(x, Wcat, bcat)


def _sc_attn(qh, kh, vh, edge_index):
    mesh = plsc.VectorSubcoreMesh(core_axis_name="c", subcore_axis_name="s")

    @functools.partial(
        pl.kernel,
        out_type=(
            jax.ShapeDtypeStruct((NC * NPAD, HALF), jnp.float32),
            jax.ShapeDtypeStruct((NC * DEN_N, 16), jnp.float32),
        ),
        mesh=mesh,
        scratch_types=[
            pltpu.VMEM((2, BLKE), jnp.int32),
            pltpu.VMEM((4, CH), jnp.int32),
            pltpu.VMEM((4, CH), jnp.int32),
            pltpu.VMEM((1, CH), jnp.int32),
            pltpu.VMEM((1, CH), jnp.int32),
            pltpu.VMEM((CH, HALF), jnp.float32),
            pltpu.VMEM((CH, HALF), jnp.float32),
            pltpu.VMEM((CH, HALF), jnp.float32),
            pltpu.VMEM((CH, HALF), jnp.float32),
            pltpu.VMEM((CH, HALF), jnp.float32),
            pltpu.VMEM((CH, HALF), jnp.float32),
            pltpu.VMEM((CH, 16), jnp.float32),
            pltpu.VMEM((8, 16), jnp.float32),
            pltpu.SemaphoreType.DMA,
            pltpu.SemaphoreType.DMA,
            pltpu.SemaphoreType.DMA,
            pltpu.SemaphoreType.DMA,
            pltpu.SemaphoreType.DMA,
            pltpu.SemaphoreType.DMA,
            pltpu.VMEM_SHARED((NPAD, HALF), jnp.float32),
            pltpu.VMEM_SHARED((DW, HALF), jnp.float32),
        ],
    )
    def attn(q_hbm, k_hbm, v_hbm, ei_hbm, acc_out, den_out,
             blk, idxA, idxB, sA, sB, kA, kB, qA, qB, vA, vB, wbuf, den_nar,
             gsemA, gsemB, vsem, ssemA, ssemB, dsem, acc_sh, den_wide):
        c = lax.axis_index("c")
        s = lax.axis_index("s")
        zero16 = jnp.zeros((16,), jnp.float32)

        v_rows = vA

        def zbody(t, carry):
            for j in range(HALF // 16):
                v_rows[t, pl.ds(16 * j, 16)] = zero16
            wbuf[t, :] = zero16
            return carry

        lax.fori_loop(0, CH, zbody, 0)

        r0 = s * ROWS_PER_TILE
        for b in range(19):
            pltpu.sync_copy(v_rows, acc_sh.at[pl.ds(r0 + b * CH, CH), :])
        pltpu.sync_copy(v_rows, acc_sh.at[pl.ds(r0 + ROWS_PER_TILE - CH, CH), :])
        w0 = s * (DW // NS)
        pltpu.sync_copy(v_rows, den_wide.at[pl.ds(w0, CH), :])
        pltpu.sync_copy(v_rows, den_wide.at[pl.ds(w0 + CH, CH), :])
        pltpu.sync_copy(v_rows, den_wide.at[pl.ds(w0 + DW // NS - CH, CH), :])
        plsc.subcore_barrier()

        lane = lax.broadcasted_iota(jnp.int32, (16,), 0)
        head_mask = lane < 4
        shuf = [lane ^ m for m in (8, 4, 2, 1)]
        crow = c * N

        def hsum(p):
            for ix in shuf:
                p = p + jnp.take_along_axis(p, ix, axis=0)
            return p

        nblk = jnp.where(s < NBLK - 15 * NS, 16, 15)
        blk0 = s * 15 + jnp.minimum(s, NBLK - 15 * NS)
        nch = nblk * CPB

        def load_block(jj):
            ebase = pl.multiple_of(blk0 * BLKE + jj * CH, 128)
            pltpu.sync_copy(ei_hbm.at[:, pl.ds(ebase, BLKE)], blk)

        def prep_idx(jj, P):
            pos = (jj % CPB) * CH
            for t in range(CH // 16):
                sl = pl.ds(pos + 16 * t, 16)
                osl = pl.ds(16 * t, 16)
                sv = blk[0, sl]
                dv = blk[1, sl]
                P[0, osl] = dv
                P[1, osl] = sv + crow
                P[2, osl] = dv + crow
                P[3, osl] = lax.shift_right_logical(dv, 3)
            return None

        def scores(kb, qb):
            def edge_a(i, cc):
                total = jnp.zeros((16,), jnp.float32)
                for hj in range(4):
                    k0 = kb[i, pl.ds(32 * hj, 16)]
                    k1 = kb[i, pl.ds(32 * hj + 16, 16)]
                    q0 = qb[i, pl.ds(32 * hj, 16)]
                    q1 = qb[i, pl.ds(32 * hj + 16, 16)]
                    p = hsum(k0 * q0 + k1 * q1)
                    total = jnp.where(lane == hj, p, total)
                wbuf[i, :] = jnp.where(head_mask,
                                       jnp.exp(total * SCALE), 0.0)
                return cc

            lax.fori_loop(0, CH, edge_a, 0)

        def den_build_fire(P, qb):
            for g in range(CH // 16):
                offv = (P[0, pl.ds(16 * g, 16)] & 7) * 16
                for jj in range(16):
                    i = 16 * g + jj
                    wrow = wbuf[i, :]
                    off = offv[jj]
                    for sl in range(8):
                        qb[i, pl.ds(16 * sl, 16)] = jnp.where(
                            off == 16 * sl, wrow, zero16)
            pltpu.async_copy(qb, den_wide.at[P.at[3]], dsem, add=True)

        def weight(vb):
            def wloop(i, cc):
                wrow = wbuf[i, :]
                for hj in range(4):
                    w = wrow[hj]
                    vr0 = vb[i, pl.ds(32 * hj, 16)]
                    vr1 = vb[i, pl.ds(32 * hj + 16, 16)]
                    vb[i, pl.ds(32 * hj, 16)] = vr0 * w
                    vb[i, pl.ds(32 * hj + 16, 16)] = vr1 * w
                return cc

            lax.fori_loop(0, CH, wloop, 0)

        load_block(0)
        prep_idx(0, idxA)
        pltpu.async_copy(k_hbm.at[idxA.at[1]], kA, gsemA)
        pltpu.async_copy(q_hbm.at[idxA.at[2]], qA, gsemA)
        for t in range(CH // 16):
            sA[0, pl.ds(16 * t, 16)] = idxA[0, pl.ds(16 * t, 16)]
            sB[0, pl.ds(16 * t, 16)] = idxA[0, pl.ds(16 * t, 16)]
        pltpu.async_copy(vA, acc_sh.at[sA.at[0]], ssemA, add=True)
        pltpu.async_copy(vA, acc_sh.at[sB.at[0]], ssemB, add=True)

        def phase(P, O, sP, kP, qP, kO, qO, vP, gsemP, gsemO, ssemP, jnext):
            pltpu.async_copy(k_hbm.at[O.at[1]], kO, gsemO)
            pltpu.async_copy(q_hbm.at[O.at[2]], qO, gsemO)
            pltpu.make_async_copy(vP, acc_sh.at[sP.at[0]], ssemP).wait()
            pltpu.async_copy(v_hbm.at[P.at[1]], vP, vsem)
            pltpu.make_async_copy(k_hbm.at[P.at[1]], kP, gsemP).wait()
            pltpu.make_async_copy(q_hbm.at[P.at[2]], qP, gsemP).wait()
            scores(kP, qP)
            den_build_fire(P, qP)
            pltpu.make_async_copy(v_hbm.at[P.at[1]], vP, vsem).wait()
            weight(vP)
            for t in range(CH // 16):
                sP[0, pl.ds(16 * t, 16)] = P[0, pl.ds(16 * t, 16)]
            pltpu.async_copy(vP, acc_sh.at[sP.at[0]], ssemP, add=True)
            pltpu.make_async_copy(qP, den_wide.at[P.at[3]], dsem).wait()

        def pair_body(j2, carry):
            jA = 2 * j2
            prep_idx(jA + 1, idxB)
            phase(idxA, idxB, sA, kA, qA, kB, qB, vA, gsemA, gsemB, ssemA,
                  jA + 1)
            jN = jA + 2

            @pl.when(jN % CPB == 0)
            def _():
                @pl.when(jN < nch)
                def _():
                    load_block(jN)

            @pl.when(jN < nch)
            def _():
                prep_idx(jN, idxA)

            phase(idxB, idxA, sB, kB, qB, kA, qA, vB, gsemB, gsemA, ssemB, jN)
            return carry

        lax.fori_loop(0, nch // 2, pair_body, 0)
        pltpu.make_async_copy(k_hbm.at[idxA.at[1]], kA, gsemA).wait()
        pltpu.make_async_copy(q_hbm.at[idxA.at[2]], qA, gsemA).wait()
        pltpu.make_async_copy(vA, acc_sh.at[sA.at[0]], ssemA).wait()
        pltpu.make_async_copy(vB, acc_sh.at[sB.at[0]], ssemB).wait()
        plsc.subcore_barrier()

        for b in range(19):
            pltpu.sync_copy(acc_sh.at[pl.ds(r0 + b * CH, CH), :], v_rows)
            pltpu.sync_copy(v_rows,
                            acc_out.at[pl.ds(c * NPAD + r0 + b * CH, CH), :])
        tb = ROWS_PER_TILE - CH
        pltpu.sync_copy(acc_sh.at[pl.ds(r0 + tb, CH), :], v_rows)
        pltpu.sync_copy(v_rows, acc_out.at[pl.ds(c * NPAD + r0 + tb, CH), :])

        for b in range(5):
            pltpu.sync_copy(den_wide.at[pl.ds(w0 + 16 * b, 16), :],
                            v_rows.at[pl.ds(0, 16), :])
            for r in range(16):
                for p8 in range(8):
                    den_nar[p8, :] = v_rows[r, pl.ds(16 * p8, 16)]
                pltpu.sync_copy(
                    den_nar,
                    den_out.at[pl.ds(c * DEN_N + 640 * s + 128 * b + 8 * r, 8), :])

    acc, den = attn(qh.reshape(NC * N, HALF), kh.reshape(NC * N, HALF),
                    vh.reshape(NC * N, HALF), edge_index)
    return acc.reshape(NC, NPAD, HALF), den.reshape(NC, DEN_N, 16)


def _out_tc(acc, den, W_out, b_out):
    BM = 1000

    def body(acc_ref, den_ref, w_ref, b_ref, o_ref):
        lane16 = lax.broadcasted_iota(jnp.int32, (BM, 16), 1)
        rows = lax.broadcasted_iota(jnp.int32, (16, HALF), 0)
        cols = lax.broadcasted_iota(jnp.int32, (16, HALF), 1)
        S = jnp.where(cols // HD == rows, 1.0, 0.0)
        out = None
        for cpart in range(NC):
            d = den_ref[cpart]
            r = jnp.where((d > 0) & (lane16 < 4), 1.0 / d, 0.0)
            rexp = jnp.dot(r, S, preferred_element_type=jnp.float32)
            a = acc_ref[cpart] * rexp
            wslice = w_ref[cpart * HALF:(cpart + 1) * HALF, :]
            t = jnp.dot(a, wslice, preferred_element_type=jnp.float32)
            out = t if out is None else out + t
        o_ref[...] = out + b_ref[...]

    return pl.pallas_call(
        body,
        grid=(N // BM,),
        in_specs=[
            pl.BlockSpec((NC, BM, HALF), lambda i: (0, i, 0)),
            pl.BlockSpec((NC, BM, 16), lambda i: (0, i, 0)),
            pl.BlockSpec((DIM, DIM), lambda i: (0, 0)),
            pl.BlockSpec((1, DIM), lambda i: (0, 0)),
        ],
        out_specs=pl.BlockSpec((BM, DIM), lambda i: (i, 0)),
        out_shape=jax.ShapeDtypeStruct((N, DIM), jnp.float32),
    )(acc, den, W_out, b_out)


def kernel(x, edge_index, W_qkv, b_qkv, W_out, b_out):
    Wr = W_qkv.reshape(DIM, H, 3 * HD)
    Wcat = jnp.concatenate([
        Wr[:, :, :HD].reshape(DIM, DIM),
        Wr[:, :, HD:2 * HD].reshape(DIM, DIM),
        Wr[:, :, 2 * HD:].reshape(DIM, DIM),
    ], axis=1)
    br = b_qkv.reshape(H, 3 * HD)
    bcat = jnp.concatenate([
        br[:, :HD].reshape(DIM),
        br[:, HD:2 * HD].reshape(DIM),
        br[:, 2 * HD:].reshape(DIM),
    ]).reshape(1, 3 * DIM)
    q, k, v = _qkv_tc(x, Wcat, bcat)
    acc, den = _sc_attn(q, k, v, edge_index)
    return _out_tc(acc, den, W_out, b_out.reshape(1, DIM))
